# baseline (speedup 1.0000x reference)
import functools

import jax
import jax.numpy as jnp
from jax.experimental import pallas as pl
from jax.experimental.pallas import tpu as pltpu

_EPS = 1e-12


def _sn_scale_kernel(w2d_ref, wk_ref, u_ref, u_out_ref, v_out_ref, ws_ref,
                     *, alpha):
    w = w2d_ref[...]
    u0 = u_ref[...]
    v_raw = jnp.dot(u0, w, preferred_element_type=jnp.float32)
    v = v_raw / (jnp.sqrt(jnp.sum(v_raw * v_raw)) + _EPS)
    u_raw = jax.lax.dot_general(v, w, (((1,), (1,)), ((), ())),
                                preferred_element_type=jnp.float32)
    ssum = jnp.sum(u_raw * u_raw)
    u_norm = jnp.sqrt(ssum)
    u_out_ref[...] = u_raw / (u_norm + _EPS)
    v_out_ref[...] = v
    sigma = ssum / (u_norm + _EPS)
    ws_ref[...] = (wk_ref[...] * (alpha / sigma)).astype(jnp.bfloat16)


def _conv_kernel(x_ref, w_ref, b_ref, o_ref, *, kh, kw, w_sp):
    nb, hw, c = x_ref.shape
    oc = o_ref.shape[2]

    def shifted(a, s):
        if s == 0:
            return a
        z = jnp.zeros((abs(s), a.shape[1]), a.dtype)
        if s > 0:
            return jnp.concatenate([a[s:, :], z], axis=0)
        return jnp.concatenate([z, a[:s, :]], axis=0)

    bm = 512 if hw % 512 == 0 and 512 % w_sp == 0 else hw

    col = jax.lax.broadcasted_iota(jnp.int32, (bm, oc), 0) % w_sp
    masks = {}
    for dw in range(kw):
        d = dw - kw // 2
        if d != 0:
            valid = (col >= -d) if d < 0 else (col < w_sp - d)
            masks[d] = jnp.where(valid, 1.0, 0.0)

    w_all = w_ref[...]
    b_row = b_ref[...]
    for b in range(nb):
        xb = x_ref[b].astype(jnp.bfloat16)

        xk = jnp.concatenate(
            [shifted(xb, (dh - kh // 2) * w_sp) for dh in range(kh)], axis=1)

        for m0 in range(0, hw, bm):
            q = jnp.dot(xk[m0:m0 + bm, :], w_all,
                        preferred_element_type=jnp.float32)
            acc = b_row + q[:, (kw // 2) * oc:(kw // 2 + 1) * oc]
            for dw in range(kw):
                d = dw - kw // 2
                if d == 0:
                    continue
                acc = acc + shifted(q[:, dw * oc:(dw + 1) * oc], d) * masks[d]
            o_ref[b, m0:m0 + bm, :] = acc


def kernel(x, w_bar, bias, u, alpha=1.6):
    N, C, H, W = x.shape
    OC, IC, KH, KW = w_bar.shape
    assert IC == C and KH % 2 == 1 and KW % 2 == 1
    Wd = IC * KH * KW
    HW = H * W

    w2d = w_bar.reshape(OC, Wd).astype(jnp.float32)
    wk = jnp.transpose(w_bar, (2, 1, 3, 0)).reshape(KH * IC, KW * OC)
    wk = wk.astype(jnp.float32)
    u_row = u.reshape(1, OC).astype(jnp.float32)

    u_new, v_new, w_scaled = pl.pallas_call(
        functools.partial(_sn_scale_kernel, alpha=float(alpha)),
        out_shape=(
            jax.ShapeDtypeStruct((1, OC), jnp.float32),
            jax.ShapeDtypeStruct((1, Wd), jnp.float32),
            jax.ShapeDtypeStruct((KH * IC, KW * OC), jnp.bfloat16),
        ),
        in_specs=[
            pl.BlockSpec((OC, Wd), lambda: (0, 0)),
            pl.BlockSpec((KH * IC, KW * OC), lambda: (0, 0)),
            pl.BlockSpec((1, OC), lambda: (0, 0)),
        ],
        out_specs=(
            pl.BlockSpec((1, OC), lambda: (0, 0)),
            pl.BlockSpec((1, Wd), lambda: (0, 0)),
            pl.BlockSpec((KH * IC, KW * OC), lambda: (0, 0)),
        ),
    )(w2d, wk, u_row)

    x_nhwc = jnp.transpose(x, (0, 2, 3, 1)).reshape(N, HW, C)
    nb = 4 if N % 4 == 0 else (2 if N % 2 == 0 else 1)
    out_nhwc = pl.pallas_call(
        functools.partial(_conv_kernel, kh=KH, kw=KW, w_sp=W),
        out_shape=jax.ShapeDtypeStruct((N, HW, OC), jnp.float32),
        grid=(N // nb,),
        in_specs=[
            pl.BlockSpec((nb, HW, C), lambda n: (n, 0, 0)),
            pl.BlockSpec((KH * IC, KW * OC), lambda n: (0, 0)),
            pl.BlockSpec((1, OC), lambda n: (0, 0)),
        ],
        out_specs=pl.BlockSpec((nb, HW, OC), lambda n: (n, 0, 0)),
        compiler_params=pltpu.CompilerParams(
            dimension_semantics=("parallel",)),
    )(x_nhwc, w_scaled, bias.reshape(1, OC).astype(jnp.float32))

    y = jnp.transpose(out_nhwc.reshape(N, H, W, OC), (0, 3, 1, 2))
    return y, u_new.reshape(OC), v_new.reshape(Wd)


# speedup vs baseline: 1.0540x; 1.0540x over previous
import functools

import jax
import jax.numpy as jnp
from jax.experimental import pallas as pl
from jax.experimental.pallas import tpu as pltpu

_EPS = 1e-12


def _conv_kernel(x_ref, w2d_ref, wk_ref, u_ref, b_ref,
                 o_ref, u_out_ref, v_out_ref, ws_ref, *, alpha, kh, kw, w_sp):
    nb, hw, c = x_ref.shape
    oc = o_ref.shape[2]

    @pl.when(pl.program_id(0) == 0)
    def _power_iter():
        w = w2d_ref[...]
        u0 = u_ref[...]
        v_raw = jnp.dot(u0, w, preferred_element_type=jnp.float32)
        v = v_raw / (jnp.sqrt(jnp.sum(v_raw * v_raw)) + _EPS)
        u_raw = jax.lax.dot_general(v, w, (((1,), (1,)), ((), ())),
                                    preferred_element_type=jnp.float32)
        ssum = jnp.sum(u_raw * u_raw)
        u_norm = jnp.sqrt(ssum)
        u_out_ref[...] = u_raw / (u_norm + _EPS)
        v_out_ref[...] = v
        sigma = ssum / (u_norm + _EPS)
        ws_ref[...] = (wk_ref[...] * (alpha / sigma)).astype(jnp.bfloat16)

    def shifted(a, s):
        if s == 0:
            return a
        z = jnp.zeros((abs(s), a.shape[1]), a.dtype)
        if s > 0:
            return jnp.concatenate([a[s:, :], z], axis=0)
        return jnp.concatenate([z, a[:s, :]], axis=0)

    bm = 512 if hw % 512 == 0 and 512 % w_sp == 0 else hw

    col = jax.lax.broadcasted_iota(jnp.int32, (bm, oc), 0) % w_sp
    masks = {}
    for dw in range(kw):
        d = dw - kw // 2
        if d != 0:
            valid = (col >= -d) if d < 0 else (col < w_sp - d)
            masks[d] = jnp.where(valid, 1.0, 0.0)

    w_all = ws_ref[...]
    b_row = b_ref[...]
    for b in range(nb):
        xb = x_ref[b].astype(jnp.bfloat16)

        xk = jnp.concatenate(
            [shifted(xb, (dh - kh // 2) * w_sp) for dh in range(kh)], axis=1)

        for m0 in range(0, hw, bm):
            q = jnp.dot(xk[m0:m0 + bm, :], w_all,
                        preferred_element_type=jnp.float32)
            acc = b_row + q[:, (kw // 2) * oc:(kw // 2 + 1) * oc]
            for dw in range(kw):
                d = dw - kw // 2
                if d == 0:
                    continue
                acc = acc + shifted(q[:, dw * oc:(dw + 1) * oc], d) * masks[d]
            o_ref[b, m0:m0 + bm, :] = acc


def kernel(x, w_bar, bias, u, alpha=1.6):
    N, C, H, W = x.shape
    OC, IC, KH, KW = w_bar.shape
    assert IC == C and KH % 2 == 1 and KW % 2 == 1
    Wd = IC * KH * KW
    HW = H * W

    w2d = w_bar.reshape(OC, Wd).astype(jnp.float32)
    wk = jnp.transpose(w_bar, (2, 1, 3, 0)).reshape(KH * IC, KW * OC)
    wk = wk.astype(jnp.float32)
    u_row = u.reshape(1, OC).astype(jnp.float32)

    x_nhwc = jnp.transpose(x, (0, 2, 3, 1)).reshape(N, HW, C)
    nb = 2 if N % 2 == 0 else 1
    out_nhwc, u_new, v_new = pl.pallas_call(
        functools.partial(_conv_kernel, alpha=float(alpha), kh=KH, kw=KW,
                          w_sp=W),
        out_shape=(
            jax.ShapeDtypeStruct((N, HW, OC), jnp.float32),
            jax.ShapeDtypeStruct((1, OC), jnp.float32),
            jax.ShapeDtypeStruct((1, Wd), jnp.float32),
        ),
        grid=(N // nb,),
        in_specs=[
            pl.BlockSpec((nb, HW, C), lambda n: (n, 0, 0)),
            pl.BlockSpec((OC, Wd), lambda n: (0, 0)),
            pl.BlockSpec((KH * IC, KW * OC), lambda n: (0, 0)),
            pl.BlockSpec((1, OC), lambda n: (0, 0)),
            pl.BlockSpec((1, OC), lambda n: (0, 0)),
        ],
        out_specs=(
            pl.BlockSpec((nb, HW, OC), lambda n: (n, 0, 0)),
            pl.BlockSpec((1, OC), lambda n: (0, 0)),
            pl.BlockSpec((1, Wd), lambda n: (0, 0)),
        ),
        scratch_shapes=[pltpu.VMEM((KH * IC, KW * OC), jnp.bfloat16)],
        compiler_params=pltpu.CompilerParams(
            dimension_semantics=("arbitrary",)),
    )(x_nhwc, w2d, wk, u_row, bias.reshape(1, OC).astype(jnp.float32))

    y = jnp.transpose(out_nhwc.reshape(N, H, W, OC), (0, 3, 1, 2))
    return y, u_new.reshape(OC), v_new.reshape(Wd)


# speedup vs baseline: 1.1137x; 1.0567x over previous
import functools

import jax
import jax.numpy as jnp
from jax.experimental import pallas as pl
from jax.experimental.pallas import tpu as pltpu

_EPS = 1e-12


def _conv_kernel(x_ref, wk_ref, u_ref, b_ref,
                 o_ref, u_out_ref, v_out_ref, ws_ref, *, alpha, kh, kw, w_sp):
    nb, hw, c = x_ref.shape
    oc = o_ref.shape[2]

    @pl.when(pl.program_id(0) == 0)
    def _power_iter():
        wk = wk_ref[...]
        u0 = u_ref[...]
        blocks = [wk[:, dw * oc:(dw + 1) * oc] for dw in range(kw)]
        v_cols = [jax.lax.dot_general(t, u0, (((1,), (1,)), ((), ())),
                                      preferred_element_type=jnp.float32)
                  for t in blocks]
        v_raw = jnp.concatenate(v_cols, axis=1)
        v = v_raw / (jnp.sqrt(jnp.sum(v_raw * v_raw)) + _EPS)
        u_raw = sum(
            jax.lax.dot_general(v[:, dw:dw + 1], blocks[dw],
                                (((0,), (0,)), ((), ())),
                                preferred_element_type=jnp.float32)
            for dw in range(kw))
        ssum = jnp.sum(u_raw * u_raw)
        u_norm = jnp.sqrt(ssum)
        u_out_ref[...] = u_raw / (u_norm + _EPS)
        v_out_ref[...] = v
        sigma = ssum / (u_norm + _EPS)
        ws_ref[...] = (wk * (alpha / sigma)).astype(jnp.bfloat16)

    def shifted(a, s):
        if s == 0:
            return a
        z = jnp.zeros((abs(s), a.shape[1]), a.dtype)
        if s > 0:
            return jnp.concatenate([a[s:, :], z], axis=0)
        return jnp.concatenate([z, a[:s, :]], axis=0)

    bm = 512 if hw % 512 == 0 and 512 % w_sp == 0 else hw

    col = jax.lax.broadcasted_iota(jnp.int32, (bm, oc), 0) % w_sp
    masks = {}
    for dw in range(kw):
        d = dw - kw // 2
        if d != 0:
            valid = (col >= -d) if d < 0 else (col < w_sp - d)
            masks[d] = jnp.where(valid, 1.0, 0.0)

    w_all = ws_ref[...]
    b_row = b_ref[...]
    for b in range(nb):
        xb = x_ref[b].astype(jnp.bfloat16)

        xk = jnp.concatenate(
            [shifted(xb, (dh - kh // 2) * w_sp) for dh in range(kh)], axis=1)

        for m0 in range(0, hw, bm):
            q = jnp.dot(xk[m0:m0 + bm, :], w_all,
                        preferred_element_type=jnp.float32)
            acc = b_row + q[:, (kw // 2) * oc:(kw // 2 + 1) * oc]
            for dw in range(kw):
                d = dw - kw // 2
                if d == 0:
                    continue
                acc = acc + shifted(q[:, dw * oc:(dw + 1) * oc], d) * masks[d]
            o_ref[b, m0:m0 + bm, :] = acc


def kernel(x, w_bar, bias, u, alpha=1.6):
    N, C, H, W = x.shape
    OC, IC, KH, KW = w_bar.shape
    assert IC == C and KH % 2 == 1 and KW % 2 == 1
    Wd = IC * KH * KW
    HW = H * W

    wk = jnp.transpose(w_bar, (2, 1, 3, 0)).reshape(KH * IC, KW * OC)
    wk = wk.astype(jnp.float32)
    u_row = u.reshape(1, OC).astype(jnp.float32)

    x_nhwc = jnp.transpose(x, (0, 2, 3, 1)).reshape(N, HW, C)
    nb = 2 if N % 2 == 0 else 1
    out_nhwc, u_new, v_q = pl.pallas_call(
        functools.partial(_conv_kernel, alpha=float(alpha), kh=KH, kw=KW,
                          w_sp=W),
        out_shape=(
            jax.ShapeDtypeStruct((N, HW, OC), jnp.float32),
            jax.ShapeDtypeStruct((1, OC), jnp.float32),
            jax.ShapeDtypeStruct((KH * IC, KW), jnp.float32),
        ),
        grid=(N // nb,),
        in_specs=[
            pl.BlockSpec((nb, HW, C), lambda n: (n, 0, 0)),
            pl.BlockSpec((KH * IC, KW * OC), lambda n: (0, 0)),
            pl.BlockSpec((1, OC), lambda n: (0, 0)),
            pl.BlockSpec((1, OC), lambda n: (0, 0)),
        ],
        out_specs=(
            pl.BlockSpec((nb, HW, OC), lambda n: (n, 0, 0)),
            pl.BlockSpec((1, OC), lambda n: (0, 0)),
            pl.BlockSpec((KH * IC, KW), lambda n: (0, 0)),
        ),
        scratch_shapes=[pltpu.VMEM((KH * IC, KW * OC), jnp.bfloat16)],
        compiler_params=pltpu.CompilerParams(
            dimension_semantics=("arbitrary",)),
    )(x_nhwc, wk, u_row, bias.reshape(1, OC).astype(jnp.float32))

    y = jnp.transpose(out_nhwc.reshape(N, H, W, OC), (0, 3, 1, 2))
    v_new = jnp.transpose(v_q.reshape(KH, IC, KW), (1, 0, 2)).reshape(Wd)
    return y, u_new.reshape(OC), v_new.reshape(Wd)
